# baseline (speedup 1.0000x reference)
import dataclasses
import functools

import jax
import jax.numpy as jnp
from jax import lax
from jax.experimental import pallas as pl
from jax.experimental.pallas import tpu as pltpu
from jax.experimental.pallas import tpu_sc as plsc

_NUM_CORES = 2
_NUM_SUBCORES = 16
_NUM_WORKERS = _NUM_CORES * _NUM_SUBCORES
_CHB = 1024


def kernel(token_ids, weight):
    B, S = token_ids.shape
    D = weight.shape[1]
    n_chunks = (B // _CHB) * S
    per_w = n_chunks // _NUM_WORKERS
    assert B % _CHB == 0 and n_chunks % _NUM_WORKERS == 0
    blocks_per_s = B // _CHB

    tids_t = token_ids.T
    mesh = plsc.VectorSubcoreMesh(core_axis_name="c", subcore_axis_name="s")

    @functools.partial(
        pl.kernel,
        mesh=mesh,
        out_type=jax.ShapeDtypeStruct((S, D, B), weight.dtype),
        compiler_params=dataclasses.replace(
            pltpu.CompilerParams(use_tc_tiling_on_sc=False),
            needs_layout_passes=False,
        ),
        scratch_types=[
            pltpu.VMEM((1, _CHB), jnp.int32),
            pltpu.VMEM((_CHB, D), jnp.float32),
            pltpu.VMEM((D, _CHB), jnp.float32),
            pltpu.SemaphoreType.DMA,
        ],
    )
    def gather_kernel(w_hbm, idx_hbm, out_hbm, idx_v, rows_v, outt_v, sem):
        wid = lax.axis_index("s") * _NUM_CORES + lax.axis_index("c")

        @pl.loop(0, per_w)
        def _(k):
            g = wid * per_w + k
            s = g // blocks_per_s
            b0 = (g % blocks_per_s) * _CHB
            pltpu.sync_copy(idx_hbm.at[pl.ds(s, 1), pl.ds(b0, _CHB)], idx_v)
            pltpu.async_copy(w_hbm.at[idx_v.at[0]], rows_v, sem).wait()

            @pl.loop(0, _CHB)
            def _(j):
                col = jnp.full((16,), j, jnp.int32)
                for h in range(D // 16):
                    v = rows_v[j, pl.ds(16 * h, 16)]
                    row = lax.iota(jnp.int32, 16) + 16 * h
                    plsc.store_scatter(outt_v, [row, col], v)

            pltpu.sync_copy(outt_v, out_hbm.at[s, :, pl.ds(b0, _CHB)])

    out = gather_kernel(weight, tids_t)
    return out.transpose(2, 0, 1)
